# baseline (speedup 1.0000x reference)
import jax
import jax.numpy as jnp
from jax.experimental import pallas as pl
from jax.experimental.pallas import tpu as pltpu

_B = 4096
_D = 1024
_K = 50
_P = 10
_SIGMA = 0.5
_ALPHA = 0.1

_RB = 256
_C = 256
_NCH = _B // _C
_RBP = 512


def _prep_body(emb_ref, w_ref, z_ref, zw_ref):
    x = emb_ref[...]
    ss = jnp.sum(x * x, axis=1, keepdims=True)
    norm = jnp.maximum(jnp.sqrt(ss), 1e-12)
    z = x / norm
    z_ref[...] = z
    zw_ref[...] = jax.lax.dot_general(
        z, w_ref[...], (((1,), (0,)), ((), ())),
        preferred_element_type=jnp.float32)[:, 0]


def _knn_body(z_row_ref, z_all_ref, knn_ref, buf_ref):
    s = pl.program_id(0)
    smw = jax.lax.rem(s, 2)
    smr = jax.lax.rem(s + 1, 2)
    zr = z_row_ref[...]
    ones_c = jnp.ones((_C, 1), jnp.float32)

    def it(t, carry):
        lo, hi = carry
        zc = z_all_ref[pl.ds(t * _C, _C), :]
        g = jax.lax.dot_general(
            zr, zc, (((1,), (1,)), ((), ())),
            preferred_element_type=jnp.float32)
        buf_ref[smw, t] = g
        mid = 0.5 * (lo + hi)
        thr = 1.0 - 0.5 * mid
        cnt = jnp.zeros((_RB, 1), jnp.float32)
        for t2 in range(_NCH):
            mk = jnp.where(buf_ref[smr, t2] >= thr, 1.0, 0.0)
            cnt = cnt + jax.lax.dot_general(
                mk, ones_c, (((1,), (0,)), ((), ())),
                preferred_element_type=jnp.float32)
        ge = cnt >= float(_K + 1)
        return jnp.where(ge, lo, mid), jnp.where(ge, mid, hi)

    lo0 = jnp.zeros((_RB, 1), jnp.float32)
    hi0 = jnp.full((_RB, 1), 4.5, jnp.float32)
    _, hi = jax.lax.fori_loop(0, _NCH, it, (lo0, hi0))
    knn_ref[...] = jnp.sqrt(hi[:, 0])


def _softplus(x):
    return jnp.maximum(x, 0.0) + jnp.log(1.0 + jnp.exp(-jnp.abs(x)))


def _final_body(knn_ref, zw_ref, noise_ref, w_ref, b_ref, out_ref):
    bval = b_ref[0]
    zw = zw_ref[...]
    gw = jax.lax.dot_general(
        noise_ref[...], w_ref[...], (((1,), (0,)), ((), ())),
        preferred_element_type=jnp.float32)
    id_loss = jnp.sum(_softplus(-(zw + bval))) / float(_B)
    iota = jax.lax.broadcasted_iota(jnp.int32, (1, _B), 1)
    v = knn_ref[...]
    ood_sum = jnp.float32(0.0)
    for p in range(_P):
        m = jnp.max(v)
        idx = jnp.min(jnp.where(v == m, iota, _B))
        hit = iota == idx
        zsel = jnp.sum(jnp.where(hit, zw, 0.0))
        ood_sum = ood_sum + _softplus(zsel + bval + _SIGMA * gw[p, 0])
        v = jnp.where(hit, -1.0, v)
    out = _ALPHA * (id_loss + ood_sum / float(_P))
    out_ref[...] = jnp.full((1, 1), out, jnp.float32)


def kernel(embeddings, labels, W, b):
    del labels
    emb = embeddings.astype(jnp.float32)
    w = W.astype(jnp.float32)

    z, zw = pl.pallas_call(
        _prep_body,
        grid=(_B // _RBP,),
        in_specs=[
            pl.BlockSpec((_RBP, _D), lambda i: (i, 0)),
            pl.BlockSpec((_D, 1), lambda i: (0, 0)),
        ],
        out_specs=[
            pl.BlockSpec((_RBP, _D), lambda i: (i, 0)),
            pl.BlockSpec((_RBP,), lambda i: (i,)),
        ],
        out_shape=[
            jax.ShapeDtypeStruct((_B, _D), jnp.float32),
            jax.ShapeDtypeStruct((_B,), jnp.float32),
        ],
    )(emb, w)

    nblk = _B // _RB
    knn = pl.pallas_call(
        _knn_body,
        grid=(nblk + 1,),
        in_specs=[
            pl.BlockSpec((_RB, _D), lambda s: (jnp.minimum(s, nblk - 1), 0)),
            pl.BlockSpec((_B, _D), lambda s: (0, 0)),
        ],
        out_specs=pl.BlockSpec((_RB,), lambda s: (jnp.maximum(s - 1, 0),)),
        out_shape=jax.ShapeDtypeStruct((_B,), jnp.float32),
        scratch_shapes=[pltpu.VMEM((2, _NCH, _RB, _C), jnp.float32)],
    )(z, z)

    noise = jax.random.normal(jax.random.key(1234), (_P, 1, _D),
                              dtype=jnp.float32).reshape(_P, _D)
    out = pl.pallas_call(
        _final_body,
        in_specs=[
            pl.BlockSpec((1, _B), lambda: (0, 0)),
            pl.BlockSpec((1, _B), lambda: (0, 0)),
            pl.BlockSpec((_P, _D), lambda: (0, 0)),
            pl.BlockSpec((_D, 1), lambda: (0, 0)),
            pl.BlockSpec(memory_space=pltpu.SMEM),
        ],
        out_specs=pl.BlockSpec((1, 1), lambda: (0, 0)),
        out_shape=jax.ShapeDtypeStruct((1, 1), jnp.float32),
    )(knn.reshape(1, _B), zw.reshape(1, _B), noise, w,
      b.astype(jnp.float32))
    return out.reshape(())
